# baseline (speedup 1.0000x reference)
import functools

import jax
import jax.numpy as jnp
from jax import lax
from jax.experimental import pallas as pl
from jax.experimental.pallas import tpu as pltpu
from jax.experimental.pallas import tpu_sc as plsc

_NUM_EMB = 1_000_000
_DIM = 32
_NW = 32
_CHUNK = 1600


def _sc_gather(flat_idx, weights, total):
    b_per_w = total // _NW
    n_chunks = b_per_w // _CHUNK
    mesh = plsc.VectorSubcoreMesh(core_axis_name="c", subcore_axis_name="s")

    @functools.partial(
        pl.kernel,
        mesh=mesh,
        out_type=jax.ShapeDtypeStruct((total, _DIM), jnp.float32),
        scratch_types=[
            pltpu.VMEM((_CHUNK,), jnp.int32),
            pltpu.VMEM((_CHUNK, _DIM), jnp.float32),
            pltpu.SemaphoreType.DMA,
        ],
        compiler_params=pltpu.CompilerParams(use_tc_tiling_on_sc=False),
    )
    def body(idx_hbm, table_hbm, out_hbm, idx_v, rows_v, sem):
        wid = lax.axis_index("s") * 2 + lax.axis_index("c")
        base = wid * b_per_w
        for i in range(n_chunks):
            off = base + i * _CHUNK
            pltpu.sync_copy(idx_hbm.at[pl.ds(off, _CHUNK)], idx_v)
            pltpu.async_copy(table_hbm.at[idx_v], rows_v, sem).wait()
            pltpu.sync_copy(rows_v, out_hbm.at[pl.ds(off, _CHUNK)])

    return body(flat_idx, weights)


def kernel(indices, weights):
    n, k = indices.shape
    total = n * k
    flat_idx = indices.reshape(total).astype(jnp.int32)
    out = _sc_gather(flat_idx, weights, total)
    return out.reshape(n, k, _DIM)


# speedup vs baseline: 1.0780x; 1.0780x over previous
import functools

import jax
import jax.numpy as jnp
from jax import lax
from jax.experimental import pallas as pl
from jax.experimental.pallas import tpu as pltpu
from jax.experimental.pallas import tpu_sc as plsc

_DIM = 32
_NJ = 50
_NI = 4096
_NW = 32
_COLS = _NJ * _NI


def _sc_embed(idx_t, table4):
    mesh = plsc.VectorSubcoreMesh(core_axis_name="c", subcore_axis_name="s")

    @functools.partial(
        pl.kernel,
        mesh=mesh,
        out_type=jax.ShapeDtypeStruct((_DIM, _COLS), jnp.float32),
        scratch_types=[
            pltpu.VMEM((_NJ, 128), jnp.int32),
            pltpu.VMEM((_NJ, 128), jnp.int32),
            pltpu.VMEM((2, 128, 128), jnp.float32),
            pltpu.VMEM((2, _DIM, 128), jnp.float32),
            pltpu.SemaphoreType.DMA,
            pltpu.SemaphoreType.DMA,
        ],
        compiler_params=pltpu.CompilerParams(needs_layout_passes=False),
    )
    def body(idx_hbm, tab_hbm, out_hbm, idx_v, r_v, g_v, st_v, gsem, osem):
        wid = lax.axis_index("s") * 2 + lax.axis_index("c")
        i0 = wid * 128
        pltpu.sync_copy(idx_hbm.at[:, pl.ds(i0, 128)], idx_v)
        lanes = lax.iota(jnp.int32, 16)

        def prep_and_fire(j, buf):
            for g in range(8):
                iv = idx_v[j, pl.ds(g * 16, 16)]
                r_v[j, pl.ds(g * 16, 16)] = lax.shift_right_logical(iv, 2)
            pltpu.async_copy(tab_hbm.at[r_v.at[j]], g_v.at[buf], gsem)

        def wait_gather(buf):
            pltpu.make_async_copy(tab_hbm.at[r_v.at[0]], g_v.at[buf], gsem).wait()

        def wait_out(buf):
            pltpu.make_async_copy(
                st_v.at[buf], out_hbm.at[:, pl.ds(0, 128)], osem
            ).wait()

        def serve(j, buf):
            for g in range(8):
                iv = idx_v[j, pl.ds(g * 16, 16)]
                soff = lax.shift_left(jnp.bitwise_and(iv, 3), 5)
                rows = lanes + (g * 16)
                for d in range(_DIM):
                    vals = plsc.load_gather(g_v.at[buf], [rows, soff + d])
                    st_v[buf, d, pl.ds(g * 16, 16)] = vals
            pltpu.async_copy(
                st_v.at[buf], out_hbm.at[:, pl.ds(j * _NI + i0, 128)], osem
            )

        prep_and_fire(0, 0)

        def step(k, carry):
            j = 2 * k
            prep_and_fire(j + 1, 1)
            wait_gather(0)

            @pl.when(k > 0)
            def _():
                wait_out(0)

            serve(j, 0)

            @pl.when(k < (_NJ // 2 - 1))
            def _():
                prep_and_fire(j + 2, 0)

            wait_gather(1)

            @pl.when(k > 0)
            def _():
                wait_out(1)

            serve(j + 1, 1)
            return carry

        lax.fori_loop(0, _NJ // 2, step, 0)
        wait_out(0)
        wait_out(1)

    return body(idx_t, table4)


def kernel(indices, weights):
    idx_t = indices.T.astype(jnp.int32)
    table4 = weights.reshape(250000, 128)
    out_t = _sc_embed(idx_t, table4)
    return out_t.reshape(_DIM, _NJ, _NI).transpose(2, 1, 0)
